# baseline (speedup 1.0000x reference)
import jax
import jax.numpy as jnp
from jax.experimental import pallas as pl
from jax.experimental.pallas import tpu as pltpu

VOXEL = jnp.array([0.2, 0.2, 0.2], dtype=jnp.float32)
RMIN = jnp.array([-51.2, -51.2, -3.2], dtype=jnp.float32)
GRID = (512, 512, 32)
VMAX = 30000
FEAT = 64
N = 80000


def _pfn_block(feats_ref, w_ref, gamma_ref, beta_ref, out_ref):
    f = feats_ref[...]
    w = w_ref[...]
    pf = jnp.dot(f, w, preferred_element_type=jnp.float32)
    pf = pf * gamma_ref[...] + beta_ref[...]
    out_ref[...] = jnp.maximum(pf, 0.0)


def _pfn(feats, W, gamma, beta):
    n = feats.shape[0]
    blk = 4000
    return pl.pallas_call(
        _pfn_block,
        grid=(n // blk,),
        in_specs=[
            pl.BlockSpec((blk, 9), lambda i: (i, 0)),
            pl.BlockSpec((9, FEAT), lambda i: (0, 0)),
            pl.BlockSpec((FEAT,), lambda i: (0,)),
            pl.BlockSpec((FEAT,), lambda i: (0,)),
        ],
        out_specs=pl.BlockSpec((blk, FEAT), lambda i: (i, 0)),
        out_shape=jax.ShapeDtypeStruct((n, FEAT), jnp.float32),
    )(feats, W, gamma, beta)


def _embed_one(points, W, gamma, beta):
    rmax = RMIN + VOXEL * jnp.array(GRID, dtype=jnp.float32)
    pts = jnp.clip(points, RMIN + 1e-4, rmax - 1e-4)
    coords = jnp.floor((pts - RMIN) / VOXEL).astype(jnp.int32)
    cx, cy, cz = coords[:, 0], coords[:, 1], coords[:, 2]
    flat = (cz * GRID[1] + cy) * GRID[0] + cx
    total = GRID[0] * GRID[1] * GRID[2]
    uniq, inv = jnp.unique(flat, size=VMAX, fill_value=total, return_inverse=True)
    inv = inv.reshape(-1)
    ones = jnp.ones((pts.shape[0],), dtype=jnp.float32)
    counts = jax.ops.segment_sum(ones, inv, num_segments=VMAX)
    denom = jnp.maximum(counts, 1.0)
    sums = jax.ops.segment_sum(pts, inv, num_segments=VMAX)
    means = sums / denom[:, None]
    f_cluster = pts - means[inv]
    vox_center = coords.astype(jnp.float32) * VOXEL + RMIN + VOXEL * 0.5
    f_center = pts - vox_center
    feats = jnp.concatenate([pts, f_cluster, f_center], axis=1)
    pf = _pfn(feats, W, gamma, beta)
    vsum = jax.ops.segment_sum(pf, inv, num_segments=VMAX)
    vfeat = vsum / denom[:, None]
    valid = counts > 0
    vfeat = jnp.where(valid[:, None], vfeat, 0.0)
    ux = uniq % GRID[0]
    uy = (uniq // GRID[0]) % GRID[1]
    uz = uniq // (GRID[0] * GRID[1])
    vcoords = jnp.stack([uz, uy, ux], axis=1).astype(jnp.int32)
    vcoords = jnp.where(valid[:, None], vcoords, -1)
    return vfeat, vcoords


def kernel(pc0s_all, pc1s_all, W, gamma, beta):
    frames = [pc0s_all, pc1s_all]
    voxel_feats_list = []
    voxel_coors_list = []
    for time_index, pc in enumerate(frames):
        ff = []
        cc = []
        for batch_index in range(pc.shape[0]):
            vfeat, vcoords = _embed_one(pc[batch_index], W, gamma, beta)
            bcol = jnp.full((vcoords.shape[0], 1), batch_index, dtype=jnp.int32)
            vc_b = jnp.concatenate([bcol, vcoords[:, ::-1]], axis=1)
            ff.append(vfeat)
            cc.append(vc_b)
        feats_sp = jnp.concatenate(ff, axis=0)
        coors_sp = jnp.concatenate(cc, axis=0)
        tcol = jnp.full((coors_sp.shape[0], 1), time_index, dtype=jnp.int32)
        coors_sp_4d = jnp.concatenate([coors_sp, tcol], axis=1)
        voxel_feats_list.append(feats_sp)
        voxel_coors_list.append(coors_sp_4d)
    all_voxel_feats = jnp.concatenate(voxel_feats_list, axis=0)
    all_coors_4d = jnp.concatenate(voxel_coors_list, axis=0)
    return all_voxel_feats, all_coors_4d
